# Initial kernel scaffold; baseline (speedup 1.0000x reference)
import functools
import jax
import jax.numpy as jnp
from jax import lax
from jax.experimental import pallas as pl
from jax.experimental.pallas import tpu as pltpu, tpu_sc as plsc

B, DIM, N, E, L = 4, 256, 2048, 32768, 5
K = 64


def _sc_probe(h, a_src, srcs, row_ptr, ea):
    mesh = plsc.VectorSubcoreMesh(core_axis_name="c", subcore_axis_name="s")

    @functools.partial(
        pl.kernel,
        out_type=[
            jax.ShapeDtypeStruct((B * N, DIM), jnp.float32),
            jax.ShapeDtypeStruct((B * N,), jnp.float32),
        ],
        mesh=mesh,
        scratch_types=[
            pltpu.VMEM((256, DIM), jnp.float32),
            pltpu.VMEM((256,), jnp.float32),
            pltpu.VMEM((K,), jnp.int32),
            pltpu.VMEM((K,), jnp.int32),
            pltpu.VMEM((K, DIM), jnp.float32),
            pltpu.VMEM((K,), jnp.float32),
            pltpu.VMEM((2048,), jnp.float32),
            pltpu.VMEM((264,), jnp.int32),
            pltpu.SemaphoreType.DMA,
        ],
    )
    def k(h_hbm, asrc_hbm, srcs_hbm, rp_hbm, ea_hbm, acc_o, den_o,
          acc_v, den_v, src_v, idx_v, rows_v, ex_v, asrc_v, rp_v, sem):
        wid = lax.axis_index("s") * 2 + lax.axis_index("c")
        b = wid // 8
        r0 = (wid % 8) * 256
        g0 = b * 2048 + r0

        zf = jnp.zeros((16,), jnp.float32)
        for jj in range(16):
            den_v[pl.ds(jj * 16, 16)] = zf

        def zero_row(i, _):
            for jj in range(DIM // 16):
                acc_v[i, pl.ds(jj * 16, 16)] = zf
            return 0

        lax.fori_loop(0, 256, zero_row, 0)

        pltpu.sync_copy(asrc_hbm.at[pl.ds(b * 2048, 2048)], asrc_v)
        pltpu.sync_copy(rp_hbm.at[pl.ds(r0, 264)], rp_v)

        e_start = rp_v[0]
        e_end = rp_v[256]
        k0 = (e_start // 8) * 8

        def chunk_body(kk, _):
            base = k0 + kk * K
            pltpu.sync_copy(srcs_hbm.at[pl.ds(base, K)], src_v)
            for g in range(K // 16):
                sv = src_v[pl.ds(g * 16, 16)]
                idx_v[pl.ds(g * 16, 16)] = sv + b * 2048
            pltpu.async_copy(h_hbm.at[idx_v], rows_v, sem).wait()
            for g in range(K // 16):
                eid = base + g * 16 + lax.iota(jnp.int32, (16,))
                valid = (eid >= e_start) & (eid < e_end)
                sv = src_v[pl.ds(g * 16, 16)]
                asv = plsc.load_gather(asrc_v, [sv])
                alpha = asv * 0.5
                alpha = jnp.where(alpha >= 0, alpha, 0.2 * alpha)
                ex = jnp.exp(alpha)
                ex = jnp.where(valid, ex, 0.0)
                ex_v[pl.ds(g * 16, 16)] = ex
                dl = jnp.clip(sv % 256, 0, 255)
                plsc.addupdate_scatter(den_v, [dl], ex, mask=valid)

            def edge_body(i, _):
                exs = ex_v[i]
                d = src_v[i] % 256
                exb = jnp.full((16,), exs, jnp.float32)
                for jj in range(DIM // 16):
                    plsc.addupdate(
                        acc_v.at[d, pl.ds(jj * 16, 16)],
                        exb * rows_v[i, pl.ds(jj * 16, 16)])
                return 0

            lax.fori_loop(0, K, edge_body, 0)
            return 0

        nchunks = (e_end - k0 + K - 1) // K
        lax.fori_loop(0, nchunks, chunk_body, 0)

        pltpu.sync_copy(acc_v, acc_o.at[pl.ds(g0, 256)])
        pltpu.sync_copy(den_v, den_o.at[pl.ds(g0, 256)])

    return k(h, a_src, srcs, row_ptr, ea)


def kernel(x, edge_attr, gamma, beta, W, att_src, att_dst, We, att_e, bias, edge_index):
    h = x.reshape(B * N, DIM)
    a_src = jnp.zeros((B * N,), jnp.float32)
    perm = jnp.argsort(edge_index[1])
    srcs = jnp.pad(edge_index[0][perm], (0, 256))
    dst_sorted = edge_index[1][perm]
    row_ptr = jnp.searchsorted(dst_sorted, jnp.arange(N + 1, dtype=jnp.int32)).astype(jnp.int32)
    row_ptr = jnp.pad(row_ptr, (0, 64))
    ea = edge_attr[:, 0]
    acc, den = _sc_probe(h, a_src, srcs, row_ptr, ea)
    out = acc.reshape(B, N, DIM)
    return jnp.transpose(out, (0, 2, 1))


# speedup vs baseline: 8.9777x; 8.9777x over previous
import functools
import jax
import jax.numpy as jnp
from jax import lax
from jax.experimental import pallas as pl
from jax.experimental.pallas import tpu as pltpu, tpu_sc as plsc

B, DIM, N, E, L = 4, 256, 2048, 32768, 5
NT = B * N
K = 64
NW = 32
RPW = N // (NW // B)
FB = DIM // 16


def _ln_kernel(x_ref, g_ref, b_ref, o_ref):
    xb = x_ref[0]
    mu = jnp.mean(xb)
    xc = xb - mu
    var = jnp.mean(xc * xc)
    inv = lax.rsqrt(var + 1e-5)
    o_ref[0] = xc * inv * g_ref[...] + b_ref[...]


def _layernorm(x, gamma, beta):
    return pl.pallas_call(
        _ln_kernel,
        grid=(B,),
        in_specs=[
            pl.BlockSpec((1, DIM, N), lambda i: (i, 0, 0)),
            pl.BlockSpec((DIM, N), lambda i: (0, 0)),
            pl.BlockSpec((DIM, N), lambda i: (0, 0)),
        ],
        out_specs=pl.BlockSpec((1, DIM, N), lambda i: (i, 0, 0)),
        out_shape=jax.ShapeDtypeStruct((B, DIM, N), jnp.float32),
    )(x, gamma, beta)


def _proj_kernel_first(h_ref, w_ref, as_ref, ad_ref, ho_ref, s_ref, d_ref):
    g = h_ref[...]
    h = jnp.dot(g, w_ref[...], preferred_element_type=jnp.float32)
    ho_ref[...] = h
    s_ref[...] = jnp.sum(h * as_ref[...], axis=1, keepdims=True)
    d_ref[...] = jnp.sum(h * ad_ref[...], axis=1, keepdims=True)


def _proj_kernel_mid(acc_ref, den_ref, bias_ref, w_ref, as_ref, ad_ref,
                     ho_ref, s_ref, d_ref):
    xin = acc_ref[...] / (den_ref[...] + 1e-16) + bias_ref[...]
    g = jax.nn.gelu(xin)
    h = jnp.dot(g, w_ref[...], preferred_element_type=jnp.float32)
    ho_ref[...] = h
    s_ref[...] = jnp.sum(h * as_ref[...], axis=1, keepdims=True)
    d_ref[...] = jnp.sum(h * ad_ref[...], axis=1, keepdims=True)


_ROWS_BLK = 1024


def _project(h_or_acc, den, bias, w, att_s, att_d, first):
    grid = (NT // _ROWS_BLK,)
    row_spec = pl.BlockSpec((_ROWS_BLK, DIM), lambda i: (i, 0))
    vec_spec = pl.BlockSpec((1, DIM), lambda i: (0, 0))
    out_specs = [
        pl.BlockSpec((_ROWS_BLK, DIM), lambda i: (i, 0)),
        pl.BlockSpec((_ROWS_BLK, 1), lambda i: (i, 0)),
        pl.BlockSpec((_ROWS_BLK, 1), lambda i: (i, 0)),
    ]
    out_shape = [
        jax.ShapeDtypeStruct((NT, DIM), jnp.float32),
        jax.ShapeDtypeStruct((NT, 1), jnp.float32),
        jax.ShapeDtypeStruct((NT, 1), jnp.float32),
    ]
    w2 = pl.BlockSpec((DIM, DIM), lambda i: (0, 0))
    if first:
        return pl.pallas_call(
            _proj_kernel_first,
            grid=grid,
            in_specs=[row_spec, w2, vec_spec, vec_spec],
            out_specs=out_specs,
            out_shape=out_shape,
        )(h_or_acc, w, att_s.reshape(1, DIM), att_d.reshape(1, DIM))
    return pl.pallas_call(
        _proj_kernel_mid,
        grid=grid,
        in_specs=[row_spec,
                  pl.BlockSpec((_ROWS_BLK, 1), lambda i: (i, 0)),
                  vec_spec, w2, vec_spec, vec_spec],
        out_specs=out_specs,
        out_shape=out_shape,
    )(h_or_acc, den.reshape(NT, 1), bias.reshape(1, DIM), w,
      att_s.reshape(1, DIM), att_d.reshape(1, DIM))


def _final_kernel(acc_ref, den_ref, bias_ref, o_ref):
    o_ref[...] = acc_ref[...] / (den_ref[...] + 1e-16) + bias_ref[...]


def _finalize(acc, den, bias):
    out2 = pl.pallas_call(
        _final_kernel,
        grid=(NT // _ROWS_BLK,),
        in_specs=[pl.BlockSpec((_ROWS_BLK, DIM), lambda i: (i, 0)),
                  pl.BlockSpec((_ROWS_BLK, 1), lambda i: (i, 0)),
                  pl.BlockSpec((1, DIM), lambda i: (0, 0))],
        out_specs=pl.BlockSpec((_ROWS_BLK, DIM), lambda i: (i, 0)),
        out_shape=jax.ShapeDtypeStruct((NT, DIM), jnp.float32),
    )(acc, den.reshape(NT, 1), bias.reshape(1, DIM))
    return jnp.transpose(out2.reshape(B, N, DIM), (0, 2, 1))


def _sc_edge_pass(h, asrc, adst, ea, srcs, dsts, perm, rp, wvec, aevec):
    mesh = plsc.VectorSubcoreMesh(core_axis_name="c", subcore_axis_name="s")

    @functools.partial(
        pl.kernel,
        out_type=[
            jax.ShapeDtypeStruct((NT, DIM), jnp.float32),
            jax.ShapeDtypeStruct((NT,), jnp.float32),
        ],
        mesh=mesh,
        scratch_types=[
            pltpu.VMEM((RPW, DIM), jnp.float32),
            pltpu.VMEM((RPW,), jnp.float32),
            pltpu.VMEM((RPW,), jnp.float32),
            pltpu.VMEM((RPW,), jnp.float32),
            pltpu.VMEM((N,), jnp.float32),
            pltpu.VMEM((RPW,), jnp.float32),
            pltpu.VMEM((E,), jnp.float32),
            pltpu.VMEM((K + 16,), jnp.int32),
            pltpu.VMEM((K + 16,), jnp.int32),
            pltpu.VMEM((K + 16,), jnp.int32),
            pltpu.VMEM((K,), jnp.int32),
            pltpu.VMEM((K, DIM), jnp.float32),
            pltpu.VMEM((K + 16,), jnp.float32),
            pltpu.VMEM((272,), jnp.int32),
            pltpu.VMEM((DIM,), jnp.float32),
            pltpu.VMEM((DIM,), jnp.float32),
            pltpu.SemaphoreType.DMA,
        ],
        compiler_params=pltpu.CompilerParams(needs_layout_passes=False),
    )
    def k(h_hbm, asrc_hbm, adst_hbm, ea_hbm, srcs_hbm, dsts_hbm, perm_hbm,
          rp_hbm, wv_hbm, aev_hbm, acc_o, den_o,
          acc_v, den_v, easum_v, c_v, asrc_v, adst_v, ea_v,
          src_v, dl_v, pm_v, idx_v, rows_v, ex_v, rp_v, wv_v, aev_v, sem):
        wid = lax.axis_index("s") * 2 + lax.axis_index("c")
        b = wid // (NW // B)
        r0 = (wid % (NW // B)) * RPW
        g0 = b * N + r0

        pltpu.sync_copy(asrc_hbm.at[pl.ds(b * N, N)], asrc_v)
        pltpu.sync_copy(adst_hbm.at[pl.ds(g0, RPW)], adst_v)
        pltpu.sync_copy(ea_hbm, ea_v)
        pltpu.sync_copy(rp_hbm.at[pl.ds(r0, 272)], rp_v)
        pltpu.sync_copy(wv_hbm, wv_v)
        pltpu.sync_copy(aev_hbm, aev_v)

        def cdot(i, acc):
            return acc + wv_v[pl.ds(i * 16, 16)] * aev_v[pl.ds(i * 16, 16)]
        c = jnp.sum(lax.fori_loop(0, FB, cdot, jnp.zeros((16,), jnp.float32)))

        def mmx(i, carry):
            mn, mx = carry
            v = ea_v[pl.ds(i * 16, 16)]
            return jnp.minimum(mn, v), jnp.maximum(mx, v)
        big = jnp.full((16,), 3.4e38, jnp.float32)
        mn, mx = lax.fori_loop(0, E // 16, mmx, (big, -big))
        cea = jnp.maximum(jnp.maximum(c * jnp.min(mn), c * jnp.max(mx)), 0.0)

        def amx(i, carry):
            return jnp.maximum(carry, asrc_v[pl.ds(i * 16, 16)])
        amax = jnp.max(lax.fori_loop(0, N // 16, amx, -big))

        zf = jnp.zeros((16,), jnp.float32)
        for jj in range(RPW // 16):
            ub = amax + cea + adst_v[pl.ds(jj * 16, 16)]
            c_v[pl.ds(jj * 16, 16)] = jnp.where(ub >= 0, ub, 0.2 * ub)
            den_v[pl.ds(jj * 16, 16)] = zf
            easum_v[pl.ds(jj * 16, 16)] = zf

        def zero_row(i, _):
            for jj in range(FB):
                acc_v[i, pl.ds(jj * 16, 16)] = zf
            return 0
        lax.fori_loop(0, RPW, zero_row, 0)

        def accum_rows(nrows):
            def row_body(i, _):
                exs = ex_v[pl.ds(i, 16)][0]
                d = dl_v[pl.ds(i, 16)][0]
                exb = jnp.full((16,), exs, jnp.float32)
                for jj in range(FB):
                    plsc.addupdate(
                        acc_v.at[d, pl.ds(jj * 16, 16)],
                        exb * rows_v[i, pl.ds(jj * 16, 16)])
                return 0
            lax.fori_loop(0, nrows, row_body, 0)

        e_start = rp_v[pl.ds(0, 16)][0]
        e_end = rp_v[pl.ds(RPW, 16)][0]
        k0 = (e_start // 8) * 8

        def chunk_body(kk, _):
            base = k0 + kk * K
            pltpu.sync_copy(srcs_hbm.at[pl.ds(base, K)], src_v.at[pl.ds(0, K)])
            pltpu.sync_copy(perm_hbm.at[pl.ds(base, K)], pm_v.at[pl.ds(0, K)])
            pltpu.sync_copy(dsts_hbm.at[pl.ds(base, K)], dl_v.at[pl.ds(0, K)])
            for g in range(K // 16):
                sv = src_v[pl.ds(g * 16, 16)]
                idx_v[pl.ds(g * 16, 16)] = sv + b * N
            pltpu.async_copy(h_hbm.at[idx_v], rows_v, sem).wait()
            for g in range(K // 16):
                sl = pl.ds(g * 16, 16)
                eid = base + g * 16 + lax.iota(jnp.int32, 16)
                valid = (eid >= e_start) & (eid < e_end)
                sv = src_v[sl]
                dl = jnp.clip(dl_v[sl] - r0, 0, RPW - 1)
                dl_v[sl] = dl
                pv = pm_v[sl]
                ea16 = plsc.load_gather(ea_v, [pv])
                asv = plsc.load_gather(asrc_v, [sv])
                adv = plsc.load_gather(adst_v, [dl])
                cv = plsc.load_gather(c_v, [dl])
                alpha = asv + adv + c * ea16
                alpha = jnp.where(alpha >= 0, alpha, 0.2 * alpha)
                ex = jnp.where(valid, jnp.exp(alpha - cv), 0.0)
                ex_v[sl] = ex
                plsc.addupdate_scatter(den_v, [dl], ex, mask=valid)
                plsc.addupdate_scatter(easum_v, [dl], ea16, mask=valid)
            accum_rows(K)
            return 0

        nchunks = (e_end - k0 + K - 1) // K
        lax.fori_loop(0, nchunks, chunk_body, 0)

        def self_body(jj, _):
            sl16 = pl.ds(jj * 16, 16)
            d16 = jj * 16 + lax.iota(jnp.int32, 16)
            rp_hi = plsc.load_gather(rp_v, [d16 + 1])
            rp_lo = plsc.load_gather(rp_v, [d16])
            cntf = (rp_hi - rp_lo).astype(jnp.float32)
            la = easum_v[sl16] / jnp.maximum(cntf, 1.0)
            asv = plsc.load_gather(asrc_v, [r0 + d16])
            alpha = asv + adst_v[sl16] + c * la
            alpha = jnp.where(alpha >= 0, alpha, 0.2 * alpha)
            ex = jnp.exp(alpha - c_v[sl16])
            plsc.addupdate(den_v.at[sl16], ex)
            ex_v[pl.ds(0, 16)] = ex
            iot = lax.iota(jnp.int32, 16)
            dl_v[pl.ds(0, 16)] = jj * 16 + iot
            pltpu.sync_copy(h_hbm.at[pl.ds(g0 + jj * 16, 16)],
                            rows_v.at[pl.ds(0, 16)])
            accum_rows(16)
            return 0
        lax.fori_loop(0, RPW // 16, self_body, 0)

        pltpu.sync_copy(acc_v, acc_o.at[pl.ds(g0, RPW)])
        pltpu.sync_copy(den_v, den_o.at[pl.ds(g0, RPW)])

    return k(h, asrc, adst, ea, srcs, dsts, perm, rp, wvec, aevec)


def kernel(x, edge_attr, gamma, beta, W, att_src, att_dst, We, att_e, bias,
           edge_index):
    dst = edge_index[1]
    perm = jnp.argsort(dst).astype(jnp.int32)
    srcs = jnp.pad(edge_index[0][perm].astype(jnp.int32), (0, 2 * K))
    dsts = jnp.pad(dst[perm].astype(jnp.int32), (0, 2 * K))
    perm_p = jnp.pad(perm, (0, 2 * K))
    rp = jnp.searchsorted(dst[perm], jnp.arange(N + 1, dtype=jnp.int32),
                          side="left").astype(jnp.int32)
    rp = jnp.pad(rp, (0, 128), constant_values=rp[-1])
    ea = edge_attr[:, 0]

    xn = _layernorm(x, gamma, beta)
    h = xn.reshape(NT, DIM)

    acc = den = None
    for l in range(L):
        if l == 0:
            h_l, asrc2, adst2 = _project(h, None, None, W[0], att_src[0],
                                         att_dst[0], first=True)
        else:
            h_l, asrc2, adst2 = _project(acc, den, bias[l - 1], W[l],
                                         att_src[l], att_dst[l], first=False)
        acc, den = _sc_edge_pass(
            h_l, asrc2.reshape(NT), adst2.reshape(NT), ea, srcs, dsts,
            perm_p, rp, We[l].reshape(DIM), att_e[l])
    return _finalize(acc, den, bias[L - 1])


# speedup vs baseline: 11.5688x; 1.2886x over previous
import functools
import jax
import jax.numpy as jnp
from jax import lax
from jax.experimental import pallas as pl
from jax.experimental.pallas import tpu as pltpu, tpu_sc as plsc

B, DIM, N, E, L = 4, 256, 2048, 32768, 5
NT = B * N
K = 64
NW = 32
RPW = N // (NW // B)
DPW = N // NW
FB = DIM // 16
EPAD = E + 4 * K


def _ln_kernel(x_ref, g_ref, b_ref, o_ref):
    xb = x_ref[0]
    mu = jnp.mean(xb)
    xc = xb - mu
    var = jnp.mean(xc * xc)
    inv = lax.rsqrt(var + 1e-5)
    o_ref[0] = xc * inv * g_ref[...] + b_ref[...]


def _layernorm(x, gamma, beta):
    return pl.pallas_call(
        _ln_kernel,
        grid=(B,),
        in_specs=[
            pl.BlockSpec((1, DIM, N), lambda i: (i, 0, 0)),
            pl.BlockSpec((DIM, N), lambda i: (0, 0)),
            pl.BlockSpec((DIM, N), lambda i: (0, 0)),
        ],
        out_specs=pl.BlockSpec((1, DIM, N), lambda i: (i, 0, 0)),
        out_shape=jax.ShapeDtypeStruct((B, DIM, N), jnp.float32),
    )(x, gamma, beta)


def _proj_kernel_first(h_ref, w_ref, as_ref, ad_ref, ho_ref, s_ref, d_ref):
    g = h_ref[...]
    h = jnp.dot(g, w_ref[...], preferred_element_type=jnp.float32)
    ho_ref[...] = h
    s_ref[...] = jnp.sum(h * as_ref[...], axis=1, keepdims=True)
    d_ref[...] = jnp.sum(h * ad_ref[...], axis=1, keepdims=True)


def _proj_kernel_mid(acc_ref, den_ref, bias_ref, w_ref, as_ref, ad_ref,
                     ho_ref, s_ref, d_ref):
    xin = acc_ref[...] / (den_ref[...] + 1e-16) + bias_ref[...]
    g = jax.nn.gelu(xin)
    h = jnp.dot(g, w_ref[...], preferred_element_type=jnp.float32)
    ho_ref[...] = h
    s_ref[...] = jnp.sum(h * as_ref[...], axis=1, keepdims=True)
    d_ref[...] = jnp.sum(h * ad_ref[...], axis=1, keepdims=True)


_ROWS_BLK = 1024


def _project(h_or_acc, den, bias, w, att_s, att_d, first):
    grid = (NT // _ROWS_BLK,)
    row_spec = pl.BlockSpec((_ROWS_BLK, DIM), lambda i: (i, 0))
    vec_spec = pl.BlockSpec((1, DIM), lambda i: (0, 0))
    out_specs = [
        pl.BlockSpec((_ROWS_BLK, DIM), lambda i: (i, 0)),
        pl.BlockSpec((_ROWS_BLK, 1), lambda i: (i, 0)),
        pl.BlockSpec((_ROWS_BLK, 1), lambda i: (i, 0)),
    ]
    out_shape = [
        jax.ShapeDtypeStruct((NT, DIM), jnp.float32),
        jax.ShapeDtypeStruct((NT, 1), jnp.float32),
        jax.ShapeDtypeStruct((NT, 1), jnp.float32),
    ]
    w2 = pl.BlockSpec((DIM, DIM), lambda i: (0, 0))
    if first:
        return pl.pallas_call(
            _proj_kernel_first,
            grid=grid,
            in_specs=[row_spec, w2, vec_spec, vec_spec],
            out_specs=out_specs,
            out_shape=out_shape,
        )(h_or_acc, w, att_s.reshape(1, DIM), att_d.reshape(1, DIM))
    return pl.pallas_call(
        _proj_kernel_mid,
        grid=grid,
        in_specs=[row_spec,
                  pl.BlockSpec((_ROWS_BLK, 1), lambda i: (i, 0)),
                  vec_spec, w2, vec_spec, vec_spec],
        out_specs=out_specs,
        out_shape=out_shape,
    )(h_or_acc, den.reshape(NT, 1), bias.reshape(1, DIM), w,
      att_s.reshape(1, DIM), att_d.reshape(1, DIM))


def _final_kernel(acc_ref, den_ref, bias_ref, o_ref):
    o_ref[...] = acc_ref[...] / (den_ref[...] + 1e-16) + bias_ref[...]


def _finalize(acc, den, bias):
    out2 = pl.pallas_call(
        _final_kernel,
        grid=(NT // _ROWS_BLK,),
        in_specs=[pl.BlockSpec((_ROWS_BLK, DIM), lambda i: (i, 0)),
                  pl.BlockSpec((_ROWS_BLK, 1), lambda i: (i, 0)),
                  pl.BlockSpec((1, DIM), lambda i: (0, 0))],
        out_specs=pl.BlockSpec((_ROWS_BLK, DIM), lambda i: (i, 0)),
        out_shape=jax.ShapeDtypeStruct((NT, DIM), jnp.float32),
    )(acc, den.reshape(NT, 1), bias.reshape(1, DIM))
    return jnp.transpose(out2.reshape(B, N, DIM), (0, 2, 1))


def _sc_prep(ea, dsts, perm, rp):
    mesh = plsc.VectorSubcoreMesh(core_axis_name="c", subcore_axis_name="s")

    @functools.partial(
        pl.kernel,
        out_type=[
            jax.ShapeDtypeStruct((EPAD,), jnp.float32),
            jax.ShapeDtypeStruct((N,), jnp.float32),
            jax.ShapeDtypeStruct((16,), jnp.float32),
            jax.ShapeDtypeStruct((16,), jnp.float32),
        ],
        mesh=mesh,
        scratch_types=[
            pltpu.VMEM((E,), jnp.float32),
            pltpu.VMEM((80,), jnp.int32),
            pltpu.VMEM((K,), jnp.int32),
            pltpu.VMEM((K + 16,), jnp.int32),
            pltpu.VMEM((K,), jnp.float32),
            pltpu.VMEM((DPW,), jnp.float32),
            pltpu.VMEM((DPW,), jnp.float32),
            pltpu.VMEM((16,), jnp.float32),
            pltpu.VMEM((16,), jnp.float32),
        ],
        compiler_params=pltpu.CompilerParams(needs_layout_passes=False),
    )
    def k(ea_hbm, dsts_hbm, perm_hbm, rp_hbm,
          easort_o, la_o, mn_o, mx_o,
          ea_v, rp_v, pm_v, dst_v, st_v, easum_v, la_v, mn_v, mx_v):
        wid = lax.axis_index("s") * 2 + lax.axis_index("c")
        d0 = wid * DPW

        pltpu.sync_copy(ea_hbm, ea_v)
        pltpu.sync_copy(rp_hbm.at[pl.ds(d0, 80)], rp_v)

        zf = jnp.zeros((16,), jnp.float32)
        for jj in range(DPW // 16):
            easum_v[pl.ds(jj * 16, 16)] = zf

        e_start = rp_v[pl.ds(0, 16)][0]
        e_end = rp_v[pl.ds(DPW, 16)][0]
        k0 = (e_start // 8) * 8

        def chunk_body(kk, _):
            base = k0 + kk * K
            pltpu.sync_copy(perm_hbm.at[pl.ds(base, K)], pm_v)
            pltpu.sync_copy(dsts_hbm.at[pl.ds(base, K)],
                            dst_v.at[pl.ds(0, K)])
            for g in range(K // 16):
                sl = pl.ds(g * 16, 16)
                eid = base + g * 16 + lax.iota(jnp.int32, 16)
                valid = (eid >= e_start) & (eid < e_end)
                ea16 = plsc.load_gather(ea_v, [pm_v[sl]])
                st_v[sl] = ea16
                dl = jnp.clip(dst_v[sl] - d0, 0, DPW - 1)
                plsc.addupdate_scatter(easum_v, [dl], ea16, mask=valid)
            pltpu.sync_copy(st_v, easort_o.at[pl.ds(base, K)])
            return 0

        nchunks = jnp.maximum((e_end - k0 + K - 1) // K, 0)
        lax.fori_loop(0, nchunks, chunk_body, 0)

        for jj in range(DPW // 16):
            d16 = jj * 16 + lax.iota(jnp.int32, 16)
            rp_hi = plsc.load_gather(rp_v, [d16 + 1])
            rp_lo = plsc.load_gather(rp_v, [d16])
            cntf = (rp_hi - rp_lo).astype(jnp.float32)
            la_v[pl.ds(jj * 16, 16)] = (easum_v[pl.ds(jj * 16, 16)]
                                        / jnp.maximum(cntf, 1.0))
        pltpu.sync_copy(la_v, la_o.at[pl.ds(d0, DPW)])

        @pl.when(wid == 0)
        def _():
            big = jnp.full((16,), 3.4e38, jnp.float32)

            def mmx(i, carry):
                mn, mx = carry
                v = ea_v[pl.ds(i * 16, 16)]
                return jnp.minimum(mn, v), jnp.maximum(mx, v)
            mn, mx = lax.fori_loop(0, E // 16, mmx, (big, -big))
            mn_v[...] = mn
            mx_v[...] = mx
            pltpu.sync_copy(mn_v, mn_o)
            pltpu.sync_copy(mx_v, mx_o)

    return k(ea, dsts, perm, rp)


def _sc_edge_pass(h, asrc, adst, easort, srcs, dsts, rp, la, mn16, mx16,
                  wvec, aevec):
    mesh = plsc.VectorSubcoreMesh(core_axis_name="c", subcore_axis_name="s")

    @functools.partial(
        pl.kernel,
        out_type=[
            jax.ShapeDtypeStruct((NT, DIM), jnp.float32),
            jax.ShapeDtypeStruct((NT,), jnp.float32),
        ],
        mesh=mesh,
        scratch_types=[
            pltpu.VMEM((RPW, DIM), jnp.float32),
            pltpu.VMEM((RPW,), jnp.float32),
            pltpu.VMEM((RPW,), jnp.float32),
            pltpu.VMEM((N,), jnp.float32),
            pltpu.VMEM((RPW,), jnp.float32),
            pltpu.VMEM((RPW,), jnp.float32),
            pltpu.VMEM((2, K), jnp.int32),
            pltpu.VMEM((2, K), jnp.int32),
            pltpu.VMEM((2, K), jnp.float32),
            pltpu.VMEM((2, K), jnp.int32),
            pltpu.VMEM((2, K, DIM), jnp.float32),
            pltpu.VMEM((K + 16,), jnp.float32),
            pltpu.VMEM((K + 16,), jnp.int32),
            pltpu.VMEM((272,), jnp.int32),
            pltpu.VMEM((DIM,), jnp.float32),
            pltpu.VMEM((DIM,), jnp.float32),
            pltpu.VMEM((16,), jnp.float32),
            pltpu.VMEM((16,), jnp.float32),
            pltpu.SemaphoreType.DMA,
            pltpu.SemaphoreType.DMA,
            pltpu.SemaphoreType.DMA,
            pltpu.SemaphoreType.DMA,
        ],
        compiler_params=pltpu.CompilerParams(needs_layout_passes=False),
    )
    def k(h_hbm, asrc_hbm, adst_hbm, ea_hbm, srcs_hbm, dsts_hbm, rp_hbm,
          la_hbm, mn_hbm, mx_hbm, wv_hbm, aev_hbm, acc_o, den_o,
          acc_v, den_v, c_v, asrc_v, adst_v, la_v,
          src_v, dst_v, ea_v, idx_v, rows_v, ex_v, dl_v,
          rp_v, wv_v, aev_v, mn_v, mx_v,
          lsem0, lsem1, gsem0, gsem1):
        wid = lax.axis_index("s") * 2 + lax.axis_index("c")
        b = wid // (NW // B)
        r0 = (wid % (NW // B)) * RPW
        g0 = b * N + r0
        lsems = [lsem0, lsem1]
        gsems = [gsem0, gsem1]

        pltpu.sync_copy(asrc_hbm.at[pl.ds(b * N, N)], asrc_v)
        pltpu.sync_copy(adst_hbm.at[pl.ds(g0, RPW)], adst_v)
        pltpu.sync_copy(la_hbm.at[pl.ds(r0, RPW)], la_v)
        pltpu.sync_copy(rp_hbm.at[pl.ds(r0, 272)], rp_v)
        pltpu.sync_copy(wv_hbm, wv_v)
        pltpu.sync_copy(aev_hbm, aev_v)
        pltpu.sync_copy(mn_hbm, mn_v)
        pltpu.sync_copy(mx_hbm, mx_v)

        def cdot(i, acc):
            return acc + wv_v[pl.ds(i * 16, 16)] * aev_v[pl.ds(i * 16, 16)]
        c = jnp.sum(lax.fori_loop(0, FB, cdot, jnp.zeros((16,), jnp.float32)))

        cea = jnp.maximum(
            jnp.maximum(c * jnp.min(mn_v[...]), c * jnp.max(mx_v[...])), 0.0)

        big = jnp.full((16,), 3.4e38, jnp.float32)

        def amx(i, carry):
            return jnp.maximum(carry, asrc_v[pl.ds(i * 16, 16)])
        amax = jnp.max(lax.fori_loop(0, N // 16, amx, -big))

        zf = jnp.zeros((16,), jnp.float32)
        for jj in range(RPW // 16):
            ub = amax + cea + adst_v[pl.ds(jj * 16, 16)]
            c_v[pl.ds(jj * 16, 16)] = jnp.where(ub >= 0, ub, 0.2 * ub)
            den_v[pl.ds(jj * 16, 16)] = zf

        def zero_row(i, _):
            for jj in range(FB):
                acc_v[i, pl.ds(jj * 16, 16)] = zf
            return 0
        lax.fori_loop(0, RPW, zero_row, 0)

        e_start = rp_v[pl.ds(0, 16)][0]
        e_end = rp_v[pl.ds(RPW, 16)][0]
        k0 = (e_start // 8) * 8

        def fire_linear(base, buf):
            sem = lsems[buf]
            pltpu.async_copy(srcs_hbm.at[pl.ds(base, K)], src_v.at[buf], sem)
            pltpu.async_copy(dsts_hbm.at[pl.ds(base, K)], dst_v.at[buf], sem)
            pltpu.async_copy(ea_hbm.at[pl.ds(base, K)], ea_v.at[buf], sem)

        def wait_linear(buf):
            sem = lsems[buf]
            pltpu.make_async_copy(srcs_hbm.at[pl.ds(0, K)], src_v.at[buf],
                                  sem).wait()
            pltpu.make_async_copy(dsts_hbm.at[pl.ds(0, K)], dst_v.at[buf],
                                  sem).wait()
            pltpu.make_async_copy(ea_hbm.at[pl.ds(0, K)], ea_v.at[buf],
                                  sem).wait()

        def fire_gather(buf):
            for g in range(K // 16):
                sl = pl.ds(g * 16, 16)
                sv = jnp.clip(src_v[buf, sl], 0, N - 1)
                idx_v[buf, sl] = sv + b * N
            pltpu.async_copy(h_hbm.at[idx_v.at[buf]], rows_v.at[buf],
                             gsems[buf])

        def wait_gather(buf):
            pltpu.make_async_copy(h_hbm.at[idx_v.at[buf]], rows_v.at[buf],
                                  gsems[buf]).wait()

        def accum_rows(buf, nrows):
            def row_body(i, _):
                d = dl_v[pl.ds(i, 16)][0]
                exs = ex_v[pl.ds(i, 16)][0]
                exb = jnp.full((16,), exs, jnp.float32)
                for jj in range(FB):
                    plsc.addupdate(
                        acc_v.at[d, pl.ds(jj * 16, 16)],
                        exb * rows_v[buf, i, pl.ds(jj * 16, 16)])
                return 0
            lax.fori_loop(0, nrows, row_body, 0)

        def process_chunk(cidx, buf):
            base = k0 + cidx * K
            for g in range(K // 16):
                sl = pl.ds(g * 16, 16)
                eid = base + g * 16 + lax.iota(jnp.int32, 16)
                valid = (eid >= e_start) & (eid < e_end)
                dl = jnp.clip(dst_v[buf, sl] - r0, 0, RPW - 1)
                dl_v[sl] = dl
                ea16 = ea_v[buf, sl]
                asv = plsc.load_gather(asrc_v, [jnp.clip(src_v[buf, sl],
                                                         0, N - 1)])
                adv = plsc.load_gather(adst_v, [dl])
                cv = plsc.load_gather(c_v, [dl])
                alpha = asv + adv + c * ea16
                alpha = jnp.where(alpha >= 0, alpha, 0.2 * alpha)
                ex = jnp.where(valid, jnp.exp(alpha - cv), 0.0)
                ex_v[sl] = ex
                plsc.addupdate_scatter(den_v, [dl], ex, mask=valid)
            accum_rows(buf, K)

        nreal = jnp.maximum((e_end - k0 + K - 1) // K, 1)
        npairs = (nreal + 1) // 2
        total = npairs * 2

        fire_linear(k0, 0)
        fire_linear(k0 + K, 1)
        wait_linear(0)
        fire_gather(0)

        def pair_body(kk, _):
            for sub in range(2):
                cidx = kk * 2 + sub
                buf = sub
                nbuf = 1 - sub
                wait_gather(buf)

                @pl.when(cidx + 1 < total)
                def _():
                    wait_linear(nbuf)
                    fire_gather(nbuf)

                process_chunk(cidx, buf)

                @pl.when(cidx + 2 < total)
                def _():
                    fire_linear(k0 + (cidx + 2) * K, buf)
            return 0

        lax.fori_loop(0, npairs, pair_body, 0)

        iot = lax.iota(jnp.int32, 16)

        def self_body(ss, _):
            d0 = ss * 64
            pltpu.sync_copy(h_hbm.at[pl.ds(g0 + d0, 64)],
                            rows_v.at[0, pl.ds(0, 64)])
            for g in range(4):
                sl_out = pl.ds(g * 16, 16)
                sl_d = pl.ds(d0 + g * 16, 16)
                d16 = d0 + g * 16 + iot
                la16 = la_v[sl_d]
                asv = plsc.load_gather(asrc_v, [r0 + d16])
                alpha = asv + adst_v[sl_d] + c * la16
                alpha = jnp.where(alpha >= 0, alpha, 0.2 * alpha)
                ex = jnp.exp(alpha - c_v[sl_d])
                plsc.addupdate(den_v.at[sl_d], ex)
                ex_v[sl_out] = ex
                dl_v[sl_out] = d16
            accum_rows(0, 64)
            return 0
        lax.fori_loop(0, RPW // 64, self_body, 0)

        pltpu.sync_copy(acc_v, acc_o.at[pl.ds(g0, RPW)])
        pltpu.sync_copy(den_v, den_o.at[pl.ds(g0, RPW)])

    return k(h, asrc, adst, easort, srcs, dsts, rp, la, mn16, mx16,
             wvec, aevec)


def kernel(x, edge_attr, gamma, beta, W, att_src, att_dst, We, att_e, bias,
           edge_index):
    dst = edge_index[1]
    perm = jnp.argsort(dst).astype(jnp.int32)
    srcs = jnp.pad(edge_index[0][perm].astype(jnp.int32), (0, EPAD - E))
    dsts = jnp.pad(dst[perm].astype(jnp.int32), (0, EPAD - E),
                   constant_values=N - 1)
    perm_p = jnp.pad(perm, (0, EPAD - E))
    rp = jnp.searchsorted(dst[perm], jnp.arange(N + 1, dtype=jnp.int32),
                          side="left").astype(jnp.int32)
    rp = jnp.pad(rp, (0, 128), constant_values=rp[-1])
    ea = edge_attr[:, 0]

    easort, la, mn16, mx16 = _sc_prep(ea, dsts, perm_p, rp)

    xn = _layernorm(x, gamma, beta)
    h = xn.reshape(NT, DIM)

    acc = den = None
    for l in range(L):
        if l == 0:
            h_l, asrc2, adst2 = _project(h, None, None, W[0], att_src[0],
                                         att_dst[0], first=True)
        else:
            h_l, asrc2, adst2 = _project(acc, den, bias[l - 1], W[l],
                                         att_src[l], att_dst[l], first=False)
        acc, den = _sc_edge_pass(
            h_l, asrc2.reshape(NT), adst2.reshape(NT), easort, srcs, dsts,
            rp, la, mn16, mx16, We[l].reshape(DIM), att_e[l])
    return _finalize(acc, den, bias[L - 1])
